# Initial kernel scaffold; baseline (speedup 1.0000x reference)
import jax
import jax.numpy as jnp
from jax.experimental import pallas as pl


def kernel(seqs, cell_emb, table, W, b):
    raise NotImplementedError("write your pallas kernel here")


# speedup vs baseline: 20.1986x; 20.1986x over previous
import jax
import jax.numpy as jnp
from jax import lax
from jax.experimental import pallas as pl


def _body(seqs_ref, cell_emb_ref, table_ref, w_ref, b_ref, out_ref, cell_ref):
    bblk = seqs_ref.shape[0]
    L = seqs_ref.shape[1]
    emb = table_ref.shape[1]

    cell = lax.dot_general(
        cell_emb_ref[...], w_ref[...],
        dimension_numbers=(((1,), (1,)), ((), ())),
        preferred_element_type=jnp.float32,
    ) + b_ref[...]
    cell_ref[...] = cell

    seq = seqs_ref[...]
    vpad = table_ref.shape[0]
    onehot = (seq[:, :, None] == lax.broadcasted_iota(jnp.int32, (1, 1, vpad), 2))
    onehot = onehot.astype(jnp.float32).reshape(bblk * L, vpad)
    emb_rows = lax.dot_general(
        onehot, table_ref[...],
        dimension_numbers=(((1,), (0,)), ((), ())),
        preferred_element_type=jnp.float32,
    ).reshape(bblk, L, emb)
    out_ref[...] = emb_rows + cell[:, None, :]


def kernel(seqs, cell_emb, table, W, b):
    B, L = seqs.shape
    vocab, emb = table.shape
    cin = cell_emb.shape[1]

    vpad = 8
    table_p = jnp.zeros((vpad, emb), jnp.float32).at[:vocab].set(table)
    b2 = b.reshape(1, emb)

    BBLK = 128
    grid = (B // BBLK,)

    total, cell = pl.pallas_call(
        _body,
        grid=grid,
        in_specs=[
            pl.BlockSpec((BBLK, L), lambda i: (i, 0)),
            pl.BlockSpec((BBLK, cin), lambda i: (i, 0)),
            pl.BlockSpec((vpad, emb), lambda i: (0, 0)),
            pl.BlockSpec((emb, cin), lambda i: (0, 0)),
            pl.BlockSpec((1, emb), lambda i: (0, 0)),
        ],
        out_specs=[
            pl.BlockSpec((BBLK, L, emb), lambda i: (i, 0, 0)),
            pl.BlockSpec((BBLK, emb), lambda i: (i, 0)),
        ],
        out_shape=[
            jax.ShapeDtypeStruct((B, L, emb), jnp.float32),
            jax.ShapeDtypeStruct((B, emb), jnp.float32),
        ],
    )(seqs, cell_emb, table_p, W, b2)
    return (total, cell)
